# Initial kernel scaffold; baseline (speedup 1.0000x reference)
import jax
import jax.numpy as jnp
from jax.experimental import pallas as pl


def kernel(x, edge_index, fc_bn_gamma, fc_bn_beta, fc_W, fc_b, conv_bn_gamma, conv_bn_beta, conv_in_W, conv_in_b, conv_out_W, conv_out_b):
    raise NotImplementedError("write your pallas kernel here")


# speedup vs baseline: 18.8808x; 18.8808x over previous
import functools

import jax
import jax.numpy as jnp
from jax import lax
from jax.experimental import pallas as pl
from jax.experimental.pallas import tpu as pltpu
from jax.experimental.pallas import tpu_sc as plsc

F32 = jnp.float32
KF = 8
NC = 2
NS = 16
NW = NC * NS
CHUNK = 128
KB = 4
IB = 8
BR = 2000


def _mesh():
    return plsc.VectorSubcoreMesh(
        core_axis_name="c", subcore_axis_name="s", num_cores=NC, num_subcores=NS
    )


def _zdims(n):
    zsl = -(-(n // NS + 1) // 8) * 8
    zch = max(d for d in range(8, 257, 8) if zsl % d == 0)
    return zsl, zch


@functools.lru_cache(maxsize=None)
def _make_degree(n, cpw):
    zsl, zch = _zdims(n)
    acc_rows = NS * zsl
    nz = zsl // zch
    nblk = cpw // KB

    nst = IB // KB

    def body(ridx, cidx, e_in, e_out, z_hbm, p_out,
             idx_r, idx_c, obi, obo, zbuf, acc, sem_s):
        c = lax.axis_index("c")
        s = lax.axis_index("s")
        wid = c * NS + s
        pltpu.sync_copy(e_in, obi)
        pltpu.sync_copy(e_out, obo)
        pltpu.sync_copy(z_hbm, zbuf)
        for q in range(nz):
            off = s * zsl + q * zch
            pltpu.sync_copy(zbuf, acc.at[pl.ds(off, zch)])
        plsc.subcore_barrier()

        @pl.loop(0, nblk)
        def _(b):
            g = b // nst
            r = b % nst

            @pl.when(r == 0)
            def _():
                pltpu.sync_copy(ridx.at[wid, pl.ds(g * IB, IB)], idx_r)
                pltpu.sync_copy(cidx.at[wid, pl.ds(g * IB, IB)], idx_c)

            base = r * KB
            hs = []
            for k in range(KB):
                hs.append(pltpu.async_copy(obi, acc.at[idx_c.at[base + k]],
                                           sem_s, add=True))
                hs.append(pltpu.async_copy(obo, acc.at[idx_r.at[base + k]],
                                           sem_s, add=True))
            for h in hs:
                h.wait()

        plsc.subcore_barrier()
        for q in range(nz):
            off = s * zsl + q * zch
            pltpu.sync_copy(acc.at[pl.ds(off, zch)], zbuf)
            pltpu.sync_copy(zbuf, p_out.at[c, pl.ds(off, zch)])

    return pl.kernel(
        body,
        out_type=jax.ShapeDtypeStruct((NC, acc_rows, KF), F32),
        mesh=_mesh(),
        compiler_params=pltpu.CompilerParams(use_tc_tiling_on_sc=False),
        scratch_types=[
            pltpu.VMEM((IB, CHUNK), jnp.int32),
            pltpu.VMEM((IB, CHUNK), jnp.int32),
            pltpu.VMEM((CHUNK, KF), F32),
            pltpu.VMEM((CHUNK, KF), F32),
            pltpu.VMEM((zch, KF), F32),
            pltpu.VMEM_SHARED((acc_rows, KF), F32),
            pltpu.SemaphoreType.DMA,
        ],
    )


@functools.lru_cache(maxsize=None)
def _make_edge_agg(n, cpw):
    zsl, zch = _zdims(n)
    acc_rows = NS * zsl
    nz = zsl // zch
    nblk = cpw // KB

    nst = IB // KB

    def body(yin, yout, ridx, cidx, z_hbm, p_out,
             idx_r, idx_c, gin, gout, zbuf, acc_in, acc_out, sem_g, sem_s):
        c = lax.axis_index("c")
        s = lax.axis_index("s")
        wid = c * NS + s
        pltpu.sync_copy(z_hbm, zbuf)
        for q in range(nz):
            off = s * zsl + q * zch
            pltpu.sync_copy(zbuf, acc_in.at[pl.ds(off, zch)])
            pltpu.sync_copy(zbuf, acc_out.at[pl.ds(off, zch)])
        plsc.subcore_barrier()

        @pl.loop(0, nblk)
        def _(b):
            g = b // nst
            r = b % nst

            @pl.when(r == 0)
            def _():
                pltpu.sync_copy(ridx.at[wid, pl.ds(g * IB, IB)], idx_r)
                pltpu.sync_copy(cidx.at[wid, pl.ds(g * IB, IB)], idx_c)

            base = r * KB
            gs = []
            for k in range(KB):
                gs.append(pltpu.async_copy(yin.at[idx_r.at[base + k]],
                                           gin.at[k], sem_g))
                gs.append(pltpu.async_copy(yout.at[idx_c.at[base + k]],
                                           gout.at[k], sem_g))
            for h in gs:
                h.wait()
            ss = []
            for k in range(KB):
                ss.append(pltpu.async_copy(gin.at[k],
                                           acc_in.at[idx_c.at[base + k]],
                                           sem_s, add=True))
                ss.append(pltpu.async_copy(gout.at[k],
                                           acc_out.at[idx_r.at[base + k]],
                                           sem_s, add=True))
            for h in ss:
                h.wait()

        plsc.subcore_barrier()
        for q in range(nz):
            off = s * zsl + q * zch
            pltpu.sync_copy(acc_in.at[pl.ds(off, zch)], zbuf)
            pltpu.sync_copy(zbuf, p_out.at[c, 0, pl.ds(off, zch)])
            pltpu.sync_copy(acc_out.at[pl.ds(off, zch)], zbuf)
            pltpu.sync_copy(zbuf, p_out.at[c, 1, pl.ds(off, zch)])

    return pl.kernel(
        body,
        out_type=jax.ShapeDtypeStruct((NC, 2, acc_rows, KF), F32),
        mesh=_mesh(),
        compiler_params=pltpu.CompilerParams(use_tc_tiling_on_sc=False),
        scratch_types=[
            pltpu.VMEM((IB, CHUNK), jnp.int32),
            pltpu.VMEM((IB, CHUNK), jnp.int32),
            pltpu.VMEM((KB, CHUNK, KF), F32),
            pltpu.VMEM((KB, CHUNK, KF), F32),
            pltpu.VMEM((zch, KF), F32),
            pltpu.VMEM_SHARED((acc_rows, KF), F32),
            pltpu.VMEM_SHARED((acc_rows, KF), F32),
            pltpu.SemaphoreType.DMA,
            pltpu.SemaphoreType.DMA,
        ],
    )


def _relu(v):
    return jnp.maximum(v, 0.0)


def _dot(a, b):
    return jnp.dot(a, b, preferred_element_type=F32,
                   precision=lax.Precision.HIGHEST)


def _prologue(x, pdeg):
    n, f0 = x.shape
    grid = n // BR

    def body(x_ref, pd_ref, st_ref, din_ref, dout_ref):
        i = pl.program_id(0)
        r = _relu(x_ref[...])

        @pl.when(i == 0)
        def _():
            st_ref[...] = jnp.zeros_like(st_ref)

        st_ref[...] += jnp.stack([jnp.sum(r, 0), jnp.sum(r * r, 0)])
        d = pd_ref[0] + pd_ref[1]
        din_ref[...] = lax.rsqrt(d[:, 0:1] + 1.0)
        dout_ref[...] = lax.rsqrt(d[:, 1:2] + 1.0)

    return pl.pallas_call(
        body,
        grid=(grid,),
        in_specs=[
            pl.BlockSpec((BR, f0), lambda i: (i, 0)),
            pl.BlockSpec((NC, BR, KF), lambda i: (0, i, 0)),
        ],
        out_specs=[
            pl.BlockSpec((2, f0), lambda i: (0, 0)),
            pl.BlockSpec((BR, 1), lambda i: (i, 0)),
            pl.BlockSpec((BR, 1), lambda i: (i, 0)),
        ],
        out_shape=[
            jax.ShapeDtypeStruct((2, f0), F32),
            jax.ShapeDtypeStruct((n, 1), F32),
            jax.ShapeDtypeStruct((n, 1), F32),
        ],
    )(x, pdeg)


def _dense1(blocks, wcat_t, bf):
    nb = len(blocks)
    n, fb = blocks[0].shape
    ftot = nb * fb
    grid = n // BR

    def body(*refs):
        bl = refs[:nb]
        w_ref, bf_ref = refs[nb], refs[nb + 1]
        h2_ref, st_ref = refs[nb + 2], refs[nb + 3]
        i = pl.program_id(0)
        w = w_ref[...]
        acc = jnp.broadcast_to(bf_ref[...], (BR, KF))
        for j in range(nb):
            acc = acc + _dot(_relu(bl[j][...]), w[j * fb:(j + 1) * fb, :])
        h2_ref[...] = acc
        r = _relu(acc)

        @pl.when(i == 0)
        def _():
            st_ref[...] = jnp.zeros_like(st_ref)

        st_ref[...] += jnp.stack([jnp.sum(r, 0), jnp.sum(r * r, 0)])

    return pl.pallas_call(
        body,
        grid=(grid,),
        in_specs=(
            [pl.BlockSpec((BR, fb), lambda i: (i, 0)) for _ in range(nb)]
            + [pl.BlockSpec((ftot, KF), lambda i: (0, 0)),
               pl.BlockSpec((1, KF), lambda i: (0, 0))]
        ),
        out_specs=[
            pl.BlockSpec((BR, KF), lambda i: (i, 0)),
            pl.BlockSpec((2, KF), lambda i: (0, 0)),
        ],
        out_shape=[
            jax.ShapeDtypeStruct((n, KF), F32),
            jax.ShapeDtypeStruct((2, KF), F32),
        ],
    )(*blocks, wcat_t, bf)


def _dense2(h2, din, dout, win_t, bin_f, wout_t, bout_f):
    n = h2.shape[0]
    npady = n + 16
    grid = n // BR

    def body(h2_ref, din_ref, dout_ref, wi_ref, bi_ref, wo_ref, bo_ref,
             yin_ref, yout_ref):
        h3 = _relu(h2_ref[...])
        yin_ref[...] = (_dot(h3, wi_ref[...]) + bi_ref[...]) * din_ref[...]
        yout_ref[...] = (_dot(h3, wo_ref[...]) + bo_ref[...]) * dout_ref[...]

    return pl.pallas_call(
        body,
        grid=(grid,),
        in_specs=[
            pl.BlockSpec((BR, KF), lambda i: (i, 0)),
            pl.BlockSpec((BR, 1), lambda i: (i, 0)),
            pl.BlockSpec((BR, 1), lambda i: (i, 0)),
            pl.BlockSpec((KF, KF), lambda i: (0, 0)),
            pl.BlockSpec((1, KF), lambda i: (0, 0)),
            pl.BlockSpec((KF, KF), lambda i: (0, 0)),
            pl.BlockSpec((1, KF), lambda i: (0, 0)),
        ],
        out_specs=[
            pl.BlockSpec((BR, KF), lambda i: (i, 0)),
            pl.BlockSpec((BR, KF), lambda i: (i, 0)),
        ],
        out_shape=[
            jax.ShapeDtypeStruct((npady, KF), F32),
            jax.ShapeDtypeStruct((npady, KF), F32),
        ],
    )(h2, din, dout, win_t, bin_f, wout_t, bout_f)


def _epilogue(p, yin, yout, din, dout, b_in, b_out):
    n = din.shape[0]
    grid = n // BR
    fo = 2 * KF

    def body(p_ref, yin_ref, yout_ref, din_ref, dout_ref, bi_ref, bo_ref,
             blk_ref, st_ref):
        i = pl.program_id(0)
        aggi = p_ref[0, 0] + p_ref[1, 0] + yin_ref[...]
        aggo = p_ref[0, 1] + p_ref[1, 1] + yout_ref[...]
        inx = din_ref[...] * aggi + bi_ref[...]
        outx = dout_ref[...] * aggo + bo_ref[...]
        blkv = jnp.concatenate([inx, outx], axis=1)
        blk_ref[...] = blkv
        r = _relu(blkv)

        @pl.when(i == 0)
        def _():
            st_ref[...] = jnp.zeros_like(st_ref)

        st_ref[...] += jnp.stack([jnp.sum(r, 0), jnp.sum(r * r, 0)])

    return pl.pallas_call(
        body,
        grid=(grid,),
        in_specs=[
            pl.BlockSpec((NC, 2, BR, KF), lambda i: (0, 0, i, 0)),
            pl.BlockSpec((BR, KF), lambda i: (i, 0)),
            pl.BlockSpec((BR, KF), lambda i: (i, 0)),
            pl.BlockSpec((BR, 1), lambda i: (i, 0)),
            pl.BlockSpec((BR, 1), lambda i: (i, 0)),
            pl.BlockSpec((1, KF), lambda i: (0, 0)),
            pl.BlockSpec((1, KF), lambda i: (0, 0)),
        ],
        out_specs=[
            pl.BlockSpec((BR, fo), lambda i: (i, 0)),
            pl.BlockSpec((2, fo), lambda i: (0, 0)),
        ],
        out_shape=[
            jax.ShapeDtypeStruct((n, fo), F32),
            jax.ShapeDtypeStruct((2, fo), F32),
        ],
    )(p, yin, yout, din, dout, b_in, b_out)


def _fold_bn(stats, gamma, beta, n):
    mean = stats[0] / n
    var = jnp.maximum(stats[1] / n - mean * mean, 0.0)
    s = gamma * lax.rsqrt(var + 1e-5)
    return s, beta - mean * s


def kernel(x, edge_index, fc_bn_gamma, fc_bn_beta, fc_W, fc_b,
           conv_bn_gamma, conv_bn_beta, conv_in_W, conv_in_b,
           conv_out_W, conv_out_b):
    n, f0 = x.shape
    e = edge_index.shape[1]
    nl = len(fc_W)

    cpw = -(-e // (NW * CHUNK * IB)) * IB
    epad = NW * cpw * CHUNK
    _, zch = _zdims(n)

    row = edge_index[0]
    col = edge_index[1]
    fill = jnp.full((epad - e,), n, jnp.int32)
    ridx = jnp.concatenate([row, fill]).reshape(NW, cpw, CHUNK)
    cidx = jnp.concatenate([col, fill]).reshape(NW, cpw, CHUNK)
    z_hbm = jnp.zeros((zch, KF), F32)
    e_in = jnp.zeros((CHUNK, KF), F32).at[:, 0].set(1.0)
    e_out = jnp.zeros((CHUNK, KF), F32).at[:, 1].set(1.0)

    pdeg = _make_degree(n, cpw)(ridx, cidx, e_in, e_out, z_hbm)
    stats0, din, dout = _prologue(x, pdeg)

    blocks = [x]
    stats = [stats0]
    for l in range(nl):
        fb = blocks[0].shape[1]
        scs, shs = [], []
        for j, st in enumerate(stats):
            sj, shj = _fold_bn(st, fc_bn_gamma[l][j * fb:(j + 1) * fb],
                               fc_bn_beta[l][j * fb:(j + 1) * fb], n)
            scs.append(sj)
            shs.append(shj)
        sc1 = jnp.concatenate(scs)
        sh1 = jnp.concatenate(shs)
        wcat_t = (fc_W[l] * sc1[None, :]).T
        bf = (fc_b[l] + fc_W[l] @ sh1)[None, :]

        h2, st2 = _dense1(blocks, wcat_t, bf)

        s2, sh2 = _fold_bn(st2, conv_bn_gamma[l], conv_bn_beta[l], n)
        win_t = (conv_in_W[l] * s2[None, :]).T
        bin_f = (conv_in_W[l] @ sh2)[None, :]
        wout_t = (conv_out_W[l] * s2[None, :]).T
        bout_f = (conv_out_W[l] @ sh2)[None, :]

        yin, yout = _dense2(h2, din, dout, win_t, bin_f, wout_t, bout_f)
        p = _make_edge_agg(n, cpw)(yin, yout, ridx, cidx, z_hbm)
        blk, stl = _epilogue(p, yin, yout, din, dout,
                             conv_in_b[l][None, :], conv_out_b[l][None, :])
        blocks.append(blk)
        stats.append(stl)

    return jnp.concatenate(blocks, axis=1)


# speedup vs baseline: 39.9899x; 2.1180x over previous
import functools

import jax
import jax.numpy as jnp
from jax import lax
from jax.experimental import pallas as pl
from jax.experimental.pallas import tpu as pltpu
from jax.experimental.pallas import tpu_sc as plsc

F32 = jnp.float32
KF = 8
NC = 2
NS = 16
NW = NC * NS
CHUNK = 128
KB = 4
IB = 8
PK = 16


def _mesh():
    return plsc.VectorSubcoreMesh(
        core_axis_name="c", subcore_axis_name="s", num_cores=NC, num_subcores=NS
    )


def _zdims(n):
    zsl = -(-(n // NS + 1) // 8) * 8
    zch = max(d for d in range(8, 257, 8) if zsl % d == 0)
    return zsl, zch


def _brp(npk):
    return max(d for d in range(8, 1025, 8) if npk % d == 0)


@functools.lru_cache(maxsize=None)
def _make_degree(n, cpw):
    zsl, zch = _zdims(n)
    acc_rows = NS * zsl
    nz = zsl // zch
    nblk = cpw // KB
    nst = IB // KB

    def body(ridx, cidx, e_in, e_out, z_hbm, p_out,
             idx_r, idx_c, obi, obo, zbuf, acc, sem_s):
        c = lax.axis_index("c")
        s = lax.axis_index("s")
        wid = c * NS + s
        pltpu.sync_copy(e_in, obi)
        pltpu.sync_copy(e_out, obo)
        pltpu.sync_copy(z_hbm, zbuf)
        for q in range(nz):
            off = s * zsl + q * zch
            pltpu.sync_copy(zbuf, acc.at[pl.ds(off, zch)])
        plsc.subcore_barrier()

        @pl.loop(0, nblk)
        def _(b):
            g = b // nst
            r = b % nst

            @pl.when(r == 0)
            def _():
                pltpu.sync_copy(ridx.at[wid, pl.ds(g * IB, IB)], idx_r)
                pltpu.sync_copy(cidx.at[wid, pl.ds(g * IB, IB)], idx_c)

            base = r * KB
            hs = []
            for k in range(KB):
                hs.append(pltpu.async_copy(obi, acc.at[idx_c.at[base + k]],
                                           sem_s, add=True))
                hs.append(pltpu.async_copy(obo, acc.at[idx_r.at[base + k]],
                                           sem_s, add=True))
            for h in hs:
                h.wait()

        plsc.subcore_barrier()
        for q in range(nz):
            off = s * zsl + q * zch
            pltpu.sync_copy(acc.at[pl.ds(off, zch)], zbuf)
            pltpu.sync_copy(zbuf, p_out.at[c, pl.ds(off, zch)])

    return pl.kernel(
        body,
        out_type=jax.ShapeDtypeStruct((NC, acc_rows, KF), F32),
        mesh=_mesh(),
        compiler_params=pltpu.CompilerParams(use_tc_tiling_on_sc=False),
        scratch_types=[
            pltpu.VMEM((IB, CHUNK), jnp.int32),
            pltpu.VMEM((IB, CHUNK), jnp.int32),
            pltpu.VMEM((CHUNK, KF), F32),
            pltpu.VMEM((CHUNK, KF), F32),
            pltpu.VMEM((zch, KF), F32),
            pltpu.VMEM_SHARED((acc_rows, KF), F32),
            pltpu.SemaphoreType.DMA,
        ],
    )


@functools.lru_cache(maxsize=None)
def _make_edge_agg(n, cpw):
    zsl, zch = _zdims(n)
    acc_rows = NS * zsl
    nz = zsl // zch
    nblk = cpw // KB
    nst = IB // KB

    def body(yin, yout, ridx, cidx, z_hbm, p_out,
             idx_r, idx_c, gin, gout, zbuf, acc_in, acc_out, sem_g, sem_s):
        c = lax.axis_index("c")
        s = lax.axis_index("s")
        wid = c * NS + s
        pltpu.sync_copy(z_hbm, zbuf)
        for q in range(nz):
            off = s * zsl + q * zch
            pltpu.sync_copy(zbuf, acc_in.at[pl.ds(off, zch)])
            pltpu.sync_copy(zbuf, acc_out.at[pl.ds(off, zch)])
        plsc.subcore_barrier()

        @pl.loop(0, nblk)
        def _(b):
            g = b // nst
            r = b % nst

            @pl.when(r == 0)
            def _():
                pltpu.sync_copy(ridx.at[wid, pl.ds(g * IB, IB)], idx_r)
                pltpu.sync_copy(cidx.at[wid, pl.ds(g * IB, IB)], idx_c)

            base = r * KB
            gs = []
            for k in range(KB):
                gs.append(pltpu.async_copy(yin.at[idx_r.at[base + k]],
                                           gin.at[k], sem_g))
                gs.append(pltpu.async_copy(yout.at[idx_c.at[base + k]],
                                           gout.at[k], sem_g))
            for h in gs:
                h.wait()
            ss = []
            for k in range(KB):
                ss.append(pltpu.async_copy(gin.at[k],
                                           acc_in.at[idx_c.at[base + k]],
                                           sem_s, add=True))
                ss.append(pltpu.async_copy(gout.at[k],
                                           acc_out.at[idx_r.at[base + k]],
                                           sem_s, add=True))
            for h in ss:
                h.wait()

        plsc.subcore_barrier()
        for q in range(nz):
            off = s * zsl + q * zch
            pltpu.sync_copy(acc_in.at[pl.ds(off, zch)], zbuf)
            pltpu.sync_copy(zbuf, p_out.at[c, 0, pl.ds(off, zch)])
            pltpu.sync_copy(acc_out.at[pl.ds(off, zch)], zbuf)
            pltpu.sync_copy(zbuf, p_out.at[c, 1, pl.ds(off, zch)])

    return pl.kernel(
        body,
        out_type=jax.ShapeDtypeStruct((NC, 2, acc_rows, KF), F32),
        mesh=_mesh(),
        compiler_params=pltpu.CompilerParams(use_tc_tiling_on_sc=False),
        scratch_types=[
            pltpu.VMEM((IB, CHUNK), jnp.int32),
            pltpu.VMEM((IB, CHUNK), jnp.int32),
            pltpu.VMEM((KB, CHUNK, KF), F32),
            pltpu.VMEM((KB, CHUNK, KF), F32),
            pltpu.VMEM((zch, KF), F32),
            pltpu.VMEM_SHARED((acc_rows, KF), F32),
            pltpu.VMEM_SHARED((acc_rows, KF), F32),
            pltpu.SemaphoreType.DMA,
            pltpu.SemaphoreType.DMA,
        ],
    )


def _relu(v):
    return jnp.maximum(v, 0.0)


def _dot(a, b):
    return jnp.dot(a, b, preferred_element_type=F32,
                   precision=lax.Precision.HIGHEST)


def _prologue(xp, pdeg_p, s0, s1):
    npk, f0p = xp.shape
    brp = _brp(npk)
    grid = npk // brp
    lw = PK * KF

    def body(x_ref, pd_ref, s0_ref, s1_ref, st_ref, din_ref, dout_ref):
        i = pl.program_id(0)
        r = _relu(x_ref[...])

        @pl.when(i == 0)
        def _():
            st_ref[...] = jnp.zeros_like(st_ref)

        st_ref[...] += jnp.stack([jnp.sum(r, 0), jnp.sum(r * r, 0)])
        d = pd_ref[0] + pd_ref[1]
        din_ref[...] = lax.rsqrt(_dot(d, s0_ref[...]) + 1.0)
        dout_ref[...] = lax.rsqrt(_dot(d, s1_ref[...]) + 1.0)

    return pl.pallas_call(
        body,
        grid=(grid,),
        in_specs=[
            pl.BlockSpec((brp, f0p), lambda i: (i, 0)),
            pl.BlockSpec((NC, brp, lw), lambda i: (0, i, 0)),
            pl.BlockSpec((lw, lw), lambda i: (0, 0)),
            pl.BlockSpec((lw, lw), lambda i: (0, 0)),
        ],
        out_specs=[
            pl.BlockSpec((2, f0p), lambda i: (0, 0)),
            pl.BlockSpec((brp, lw), lambda i: (i, 0)),
            pl.BlockSpec((brp, lw), lambda i: (i, 0)),
        ],
        out_shape=[
            jax.ShapeDtypeStruct((2, f0p), F32),
            jax.ShapeDtypeStruct((npk, lw), F32),
            jax.ShapeDtypeStruct((npk, lw), F32),
        ],
    )(xp, pdeg_p, s0, s1)


def _dense1(blocks, wbd, bf_tile, nvalid):
    nb = len(blocks)
    npk, fbp = blocks[0].shape
    brp = _brp(npk)
    grid = npk // brp
    lw = PK * KF

    def body(*refs):
        bl = refs[:nb]
        w_ref, bf_ref = refs[nb], refs[nb + 1]
        h2_ref, st_ref = refs[nb + 2], refs[nb + 3]
        i = pl.program_id(0)
        w = w_ref[...]
        acc = jnp.broadcast_to(bf_ref[...], (brp, lw))
        for j in range(nb):
            acc = acc + _dot(_relu(bl[j][...]), w[j * fbp:(j + 1) * fbp, :])
        h2_ref[...] = acc
        rid = i * brp + lax.broadcasted_iota(jnp.int32, (brp, 1), 0)
        r = jnp.where(rid < nvalid, _relu(acc), 0.0)

        @pl.when(i == 0)
        def _():
            st_ref[...] = jnp.zeros_like(st_ref)

        st_ref[...] += jnp.stack([jnp.sum(r, 0), jnp.sum(r * r, 0)])

    return pl.pallas_call(
        body,
        grid=(grid,),
        in_specs=(
            [pl.BlockSpec((brp, fbp), lambda i: (i, 0)) for _ in range(nb)]
            + [pl.BlockSpec((nb * fbp, lw), lambda i: (0, 0)),
               pl.BlockSpec((1, lw), lambda i: (0, 0))]
        ),
        out_specs=[
            pl.BlockSpec((brp, lw), lambda i: (i, 0)),
            pl.BlockSpec((2, lw), lambda i: (0, 0)),
        ],
        out_shape=[
            jax.ShapeDtypeStruct((npk, lw), F32),
            jax.ShapeDtypeStruct((2, lw), F32),
        ],
    )(*blocks, wbd, bf_tile)


def _dense2(h2p, din_p, dout_p, bdwin, bin_t, bdwout, bout_t):
    npk, lw = h2p.shape
    brp = _brp(npk)
    grid = npk // brp

    def body(h2_ref, din_ref, dout_ref, wi_ref, bi_ref, wo_ref, bo_ref,
             yin_ref, yout_ref):
        h3 = _relu(h2_ref[...])
        yin_ref[...] = (_dot(h3, wi_ref[...]) + bi_ref[...]) * din_ref[...]
        yout_ref[...] = (_dot(h3, wo_ref[...]) + bo_ref[...]) * dout_ref[...]

    return pl.pallas_call(
        body,
        grid=(grid,),
        in_specs=[
            pl.BlockSpec((brp, lw), lambda i: (i, 0)),
            pl.BlockSpec((brp, lw), lambda i: (i, 0)),
            pl.BlockSpec((brp, lw), lambda i: (i, 0)),
            pl.BlockSpec((lw, lw), lambda i: (0, 0)),
            pl.BlockSpec((1, lw), lambda i: (0, 0)),
            pl.BlockSpec((lw, lw), lambda i: (0, 0)),
            pl.BlockSpec((1, lw), lambda i: (0, 0)),
        ],
        out_specs=[
            pl.BlockSpec((brp, lw), lambda i: (i, 0)),
            pl.BlockSpec((brp, lw), lambda i: (i, 0)),
        ],
        out_shape=[
            jax.ShapeDtypeStruct((npk, lw), F32),
            jax.ShapeDtypeStruct((npk, lw), F32),
        ],
    )(h2p, din_p, dout_p, bdwin, bin_t, bdwout, bout_t)


def _epilogue(pp, yinp, youtp, din_p, dout_p, bin_t, bout_t, pin, pout, nvalid):
    npk, lw = yinp.shape
    brp = _brp(npk)
    grid = npk // brp
    fop = 2 * lw

    def body(p_ref, yin_ref, yout_ref, din_ref, dout_ref, bi_ref, bo_ref,
             pin_ref, pout_ref, blk_ref, st_ref):
        i = pl.program_id(0)
        aggi = p_ref[0, 0] + p_ref[1, 0] + yin_ref[...]
        aggo = p_ref[0, 1] + p_ref[1, 1] + yout_ref[...]
        inx = din_ref[...] * aggi + bi_ref[...]
        outx = dout_ref[...] * aggo + bo_ref[...]
        blkv = _dot(inx, pin_ref[...]) + _dot(outx, pout_ref[...])
        blk_ref[...] = blkv
        rid = i * brp + lax.broadcasted_iota(jnp.int32, (brp, 1), 0)
        r = jnp.where(rid < nvalid, _relu(blkv), 0.0)

        @pl.when(i == 0)
        def _():
            st_ref[...] = jnp.zeros_like(st_ref)

        st_ref[...] += jnp.stack([jnp.sum(r, 0), jnp.sum(r * r, 0)])

    return pl.pallas_call(
        body,
        grid=(grid,),
        in_specs=[
            pl.BlockSpec((NC, 2, brp, lw), lambda i: (0, 0, i, 0)),
            pl.BlockSpec((brp, lw), lambda i: (i, 0)),
            pl.BlockSpec((brp, lw), lambda i: (i, 0)),
            pl.BlockSpec((brp, lw), lambda i: (i, 0)),
            pl.BlockSpec((brp, lw), lambda i: (i, 0)),
            pl.BlockSpec((1, lw), lambda i: (0, 0)),
            pl.BlockSpec((1, lw), lambda i: (0, 0)),
            pl.BlockSpec((lw, fop), lambda i: (0, 0)),
            pl.BlockSpec((lw, fop), lambda i: (0, 0)),
        ],
        out_specs=[
            pl.BlockSpec((brp, fop), lambda i: (i, 0)),
            pl.BlockSpec((2, fop), lambda i: (0, 0)),
        ],
        out_shape=[
            jax.ShapeDtypeStruct((npk, fop), F32),
            jax.ShapeDtypeStruct((2, fop), F32),
        ],
    )(pp, yinp, youtp, din_p, dout_p, bin_t, bout_t, pin, pout)


def _fold_bn(stats_p, fb, gamma, beta, n):
    sums = stats_p[0].reshape(PK, fb).sum(0)
    sqs = stats_p[1].reshape(PK, fb).sum(0)
    mean = sums / n
    var = jnp.maximum(sqs / n - mean * mean, 0.0)
    s = gamma * lax.rsqrt(var + 1e-5)
    return s, beta - mean * s


def kernel(x, edge_index, fc_bn_gamma, fc_bn_beta, fc_W, fc_b,
           conv_bn_gamma, conv_bn_beta, conv_in_W, conv_in_b,
           conv_out_W, conv_out_b):
    n, f0 = x.shape
    e = edge_index.shape[1]
    nl = len(fc_W)

    cpw = -(-e // (NW * CHUNK * IB)) * IB
    epad = NW * cpw * CHUNK
    zsl, zch = _zdims(n)
    acc_rows = NS * zsl
    npk = acc_rows // PK
    nvalid = n // PK
    lw = PK * KF
    eye = jnp.eye(PK, dtype=F32)

    row = edge_index[0]
    col = edge_index[1]
    fill = jnp.full((epad - e,), n, jnp.int32)
    ridx = jnp.concatenate([row, fill]).reshape(NW, cpw, CHUNK)
    cidx = jnp.concatenate([col, fill]).reshape(NW, cpw, CHUNK)
    z_hbm = jnp.zeros((zch, KF), F32)
    e_in = jnp.zeros((CHUNK, KF), F32).at[:, 0].set(1.0)
    e_out = jnp.zeros((CHUNK, KF), F32).at[:, 1].set(1.0)

    t0 = jnp.zeros((KF, KF), F32).at[0, :].set(1.0)
    t1 = jnp.zeros((KF, KF), F32).at[1, :].set(1.0)
    s0 = jnp.kron(eye, t0)
    s1 = jnp.kron(eye, t1)
    pin = jnp.kron(eye, jnp.eye(KF, 2 * KF, 0, dtype=F32))
    pout = jnp.kron(eye, jnp.eye(KF, 2 * KF, KF, dtype=F32))

    pdeg = _make_degree(n, cpw)(ridx, cidx, e_in, e_out, z_hbm)
    pdeg_p = pdeg.reshape(NC, npk, lw)

    xp = jnp.concatenate(
        [x, jnp.zeros((npk * PK - n, f0), F32)]).reshape(npk, PK * f0)
    stats0, din_p, dout_p = _prologue(xp, pdeg_p, s0, s1)

    blocks = [xp]
    stats = [stats0]
    for l in range(nl):
        fb = 2 * KF
        scs, shs = [], []
        for j, st in enumerate(stats):
            sj, shj = _fold_bn(st, fb, fc_bn_gamma[l][j * fb:(j + 1) * fb],
                               fc_bn_beta[l][j * fb:(j + 1) * fb], n)
            scs.append(sj)
            shs.append(shj)
        sc1 = jnp.concatenate(scs)
        sh1 = jnp.concatenate(shs)
        wcat_t = (fc_W[l] * sc1[None, :]).T
        bf = fc_b[l] + fc_W[l] @ sh1
        wbd = jnp.concatenate(
            [jnp.kron(eye, wcat_t[j * fb:(j + 1) * fb, :])
             for j in range(len(stats))], axis=0)
        bf_tile = jnp.tile(bf[None, :], (1, PK))

        h2p, st2 = _dense1(blocks, wbd, bf_tile, nvalid)

        s2, sh2 = _fold_bn(st2, KF, conv_bn_gamma[l], conv_bn_beta[l], n)
        bdwin = jnp.kron(eye, (conv_in_W[l] * s2[None, :]).T)
        bin_t = jnp.tile((conv_in_W[l] @ sh2)[None, :], (1, PK))
        bdwout = jnp.kron(eye, (conv_out_W[l] * s2[None, :]).T)
        bout_t = jnp.tile((conv_out_W[l] @ sh2)[None, :], (1, PK))

        yinp, youtp = _dense2(h2p, din_p, dout_p, bdwin, bin_t, bdwout, bout_t)
        p = _make_edge_agg(n, cpw)(yinp.reshape(acc_rows, KF),
                                   youtp.reshape(acc_rows, KF),
                                   ridx, cidx, z_hbm)
        pp = p.reshape(NC, 2, npk, lw)
        blk, stl = _epilogue(pp, yinp, youtp, din_p, dout_p,
                             jnp.tile(conv_in_b[l][None, :], (1, PK)),
                             jnp.tile(conv_out_b[l][None, :], (1, PK)),
                             pin, pout, nvalid)
        blocks.append(blk)
        stats.append(stl)

    return jnp.concatenate(
        [b.reshape(npk * PK, -1)[:n] for b in blocks], axis=1)
